# baseline (speedup 1.0000x reference)
import functools

import jax
import jax.numpy as jnp
from jax import lax
from jax.experimental import pallas as pl
from jax.experimental.pallas import tpu as pltpu
from jax.experimental.pallas import tpu_sc as plsc

BATCH = 16384
EMBED = 64
STYLE = 128
N_EMO = 32
N_SPK = 100000

NC = 2
NS = 16
NW = NC * NS
B_PER_W = BATCH // NW
IDX_CHUNK = 64
N_CHUNKS = B_PER_W // IDX_CHUNK

BB = 8192
N_BLK = BATCH // BB

TB = 25600
N_TBLK = -(-N_SPK // TB)


def _project_body(tt_ref, w1s_ref, out_ref):
    out_ref[...] = lax.dot_general(
        tt_ref[...], w1s_ref[...], (((0,), (0,)), ((), ())),
        preferred_element_type=jnp.float32)


def _project_table(tableT, w1sT):
    return pl.pallas_call(
        _project_body,
        grid=(N_TBLK,),
        in_specs=[
            pl.BlockSpec((EMBED, TB), lambda i: (0, i)),
            pl.BlockSpec((EMBED, STYLE), lambda i: (0, 0)),
        ],
        out_specs=pl.BlockSpec((TB, STYLE), lambda i: (i, 0)),
        out_shape=jax.ShapeDtypeStruct((N_TBLK * TB, STYLE), jnp.float32),
        compiler_params=pltpu.CompilerParams(
            vmem_limit_bytes=56 * 1024 * 1024),
    )(tableT, w1sT)


def _sc_gather_body(idx_hbm, table_hbm, out_hbm, idx_v, rows_v, gsem, osem):
    wid = lax.axis_index("s") * NC + lax.axis_index("c")
    base = wid * B_PER_W
    pltpu.sync_copy(idx_hbm.at[pl.ds(base, B_PER_W)], idx_v)
    copies = [
        pltpu.async_copy(
            table_hbm.at[idx_v.at[pl.ds(j * IDX_CHUNK, IDX_CHUNK)]],
            rows_v.at[pl.ds(j * IDX_CHUNK, IDX_CHUNK)],
            gsem,
        )
        for j in range(N_CHUNKS)
    ]
    outs = []
    for j in range(N_CHUNKS):
        copies[j].wait()
        sl = pl.ds(j * IDX_CHUNK, IDX_CHUNK)
        outs.append(
            pltpu.async_copy(
                rows_v.at[sl], out_hbm.at[pl.ds(base + j * IDX_CHUNK,
                                                IDX_CHUNK)], osem))
    for o in outs:
        o.wait()


@functools.lru_cache(maxsize=None)
def _make_spk_gather():
    return pl.kernel(
        _sc_gather_body,
        out_type=jax.ShapeDtypeStruct((BATCH, STYLE), jnp.float32),
        mesh=plsc.VectorSubcoreMesh(core_axis_name="c", subcore_axis_name="s",
                                    num_cores=NC, num_subcores=NS),
        scratch_types=[
            pltpu.VMEM((B_PER_W,), jnp.int32),
            pltpu.VMEM((B_PER_W, STYLE), jnp.float32),
            pltpu.SemaphoreType.DMA,
            pltpu.SemaphoreType.DMA,
        ],
        compiler_params=pltpu.CompilerParams(use_tc_tiling_on_sc=False),
    )


def _mlp_body(spk_ref, eid_ref, emo_ref, w1e_ref, b1_ref, w2_ref, b2_ref,
              out_ref):
    eid = eid_ref[...]
    onehot = (eid[:, None] == lax.broadcasted_iota(jnp.int32, (BB, N_EMO), 1)
              ).astype(jnp.float32)
    emo_proj = jnp.dot(emo_ref[...], w1e_ref[...],
                       preferred_element_type=jnp.float32) + b1_ref[...]
    h = spk_ref[...] + jnp.dot(onehot, emo_proj,
                               preferred_element_type=jnp.float32)
    h = jnp.maximum(h, 0.0)
    out_ref[...] = jnp.dot(h, w2_ref[...],
                           preferred_element_type=jnp.float32) + b2_ref[...]


def _mlp(spk, eid, emo_table, w1eT, b1, w2T, b2):
    return pl.pallas_call(
        _mlp_body,
        grid=(N_BLK,),
        in_specs=[
            pl.BlockSpec((BB, STYLE), lambda i: (i, 0)),
            pl.BlockSpec((BB,), lambda i: (i,)),
            pl.BlockSpec((N_EMO, EMBED), lambda i: (0, 0)),
            pl.BlockSpec((EMBED, STYLE), lambda i: (0, 0)),
            pl.BlockSpec((1, STYLE), lambda i: (0, 0)),
            pl.BlockSpec((STYLE, STYLE), lambda i: (0, 0)),
            pl.BlockSpec((1, STYLE), lambda i: (0, 0)),
        ],
        out_specs=pl.BlockSpec((BB, STYLE), lambda i: (i, 0)),
        out_shape=jax.ShapeDtypeStruct((BATCH, STYLE), jnp.float32),
    )(spk, eid, emo_table, w1eT, b1, w2T, b2)


def kernel(speaker_id, emotion_id, spk_table, emo_table, W1, b1, W2, b2):
    sid = speaker_id.astype(jnp.int32)
    eid = emotion_id.astype(jnp.int32)
    w1sT = W1[:, :EMBED].T
    w1eT = W1[:, EMBED:].T
    proj = _project_table(spk_table.T, w1sT)
    spk = _make_spk_gather()(sid, proj)
    out = _mlp(spk, eid, emo_table, w1eT,
               b1.reshape(1, STYLE), W2.T, b2.reshape(1, STYLE))
    return out


# speedup vs baseline: 1.0203x; 1.0203x over previous
import functools

import jax
import jax.numpy as jnp
from jax import lax
from jax.experimental import pallas as pl
from jax.experimental.pallas import tpu as pltpu
from jax.experimental.pallas import tpu_sc as plsc

BATCH = 16384
EMBED = 64
STYLE = 128
N_EMO = 32
N_SPK = 100000

NC = 2
NS = 16
NW = NC * NS
B_PER_W = BATCH // NW
IDX_CHUNK = 128
N_CHUNKS = B_PER_W // IDX_CHUNK

BB = 8192
N_BLK = BATCH // BB

TB = 25600
N_TBLK = -(-N_SPK // TB)


def _project_body(tt_ref, w1_ref, out_ref):
    w1s = w1_ref[...][:, :EMBED]
    out_ref[...] = lax.dot_general(
        tt_ref[...], w1s, (((0,), (1,)), ((), ())),
        preferred_element_type=jnp.float32)


def _project_table(tableT, w1):
    return pl.pallas_call(
        _project_body,
        grid=(N_TBLK,),
        in_specs=[
            pl.BlockSpec((EMBED, TB), lambda i: (0, i)),
            pl.BlockSpec((STYLE, 2 * EMBED), lambda i: (0, 0)),
        ],
        out_specs=pl.BlockSpec((TB, STYLE), lambda i: (i, 0)),
        out_shape=jax.ShapeDtypeStruct((N_TBLK * TB, STYLE), jnp.float32),
        compiler_params=pltpu.CompilerParams(
            vmem_limit_bytes=56 * 1024 * 1024),
    )(tableT, w1)


def _sc_gather_body(idx_hbm, table_hbm, out_hbm, idx_v, rows_v, gsem, osem):
    wid = lax.axis_index("s") * NC + lax.axis_index("c")
    base = wid * B_PER_W
    pltpu.sync_copy(idx_hbm.at[pl.ds(base, B_PER_W)], idx_v)
    copies = [
        pltpu.async_copy(
            table_hbm.at[idx_v.at[pl.ds(j * IDX_CHUNK, IDX_CHUNK)]],
            rows_v.at[pl.ds(j * IDX_CHUNK, IDX_CHUNK)],
            gsem,
        )
        for j in range(N_CHUNKS)
    ]
    outs = []
    for j in range(N_CHUNKS):
        copies[j].wait()
        sl = pl.ds(j * IDX_CHUNK, IDX_CHUNK)
        outs.append(
            pltpu.async_copy(
                rows_v.at[sl], out_hbm.at[pl.ds(base + j * IDX_CHUNK,
                                                IDX_CHUNK)], osem))
    for o in outs:
        o.wait()


@functools.lru_cache(maxsize=None)
def _make_spk_gather():
    return pl.kernel(
        _sc_gather_body,
        out_type=jax.ShapeDtypeStruct((BATCH, STYLE), jnp.float32),
        mesh=plsc.VectorSubcoreMesh(core_axis_name="c", subcore_axis_name="s",
                                    num_cores=NC, num_subcores=NS),
        scratch_types=[
            pltpu.VMEM((B_PER_W,), jnp.int32),
            pltpu.VMEM((B_PER_W, STYLE), jnp.float32),
            pltpu.SemaphoreType.DMA,
            pltpu.SemaphoreType.DMA,
        ],
        compiler_params=pltpu.CompilerParams(use_tc_tiling_on_sc=False),
    )


def _mlp_body(spk_ref, eid_ref, emo_ref, w1_ref, b1_ref, w2_ref, b2_ref,
              out_ref):
    eid = eid_ref[...]
    onehot = (eid[:, None] == lax.broadcasted_iota(jnp.int32, (BB, N_EMO), 1)
              ).astype(jnp.float32)
    w1e = w1_ref[...][:, EMBED:]
    emo_proj = lax.dot_general(
        emo_ref[...], w1e, (((1,), (1,)), ((), ())),
        preferred_element_type=jnp.float32) + b1_ref[...][None, :]
    h = spk_ref[...] + jnp.dot(onehot, emo_proj,
                               preferred_element_type=jnp.float32)
    h = jnp.maximum(h, 0.0)
    out_ref[...] = lax.dot_general(
        h, w2_ref[...], (((1,), (1,)), ((), ())),
        preferred_element_type=jnp.float32) + b2_ref[...][None, :]


def _mlp(spk, eid, emo_table, w1, b1, w2, b2):
    return pl.pallas_call(
        _mlp_body,
        grid=(N_BLK,),
        in_specs=[
            pl.BlockSpec((BB, STYLE), lambda i: (i, 0)),
            pl.BlockSpec((BB,), lambda i: (i,)),
            pl.BlockSpec((N_EMO, EMBED), lambda i: (0, 0)),
            pl.BlockSpec((STYLE, 2 * EMBED), lambda i: (0, 0)),
            pl.BlockSpec((STYLE,), lambda i: (0,)),
            pl.BlockSpec((STYLE, STYLE), lambda i: (0, 0)),
            pl.BlockSpec((STYLE,), lambda i: (0,)),
        ],
        out_specs=pl.BlockSpec((BB, STYLE), lambda i: (i, 0)),
        out_shape=jax.ShapeDtypeStruct((BATCH, STYLE), jnp.float32),
    )(spk, eid, emo_table, w1, b1, w2, b2)


def kernel(speaker_id, emotion_id, spk_table, emo_table, W1, b1, W2, b2):
    sid = speaker_id.astype(jnp.int32)
    eid = emotion_id.astype(jnp.int32)
    proj = _project_table(spk_table.T, W1)
    spk = _make_spk_gather()(sid, proj)
    out = _mlp(spk, eid, emo_table, W1, b1, W2, b2)
    return out
